# Initial kernel scaffold; baseline (speedup 1.0000x reference)
import jax
import jax.numpy as jnp
from jax.experimental import pallas as pl


def kernel(dist, table):
    raise NotImplementedError("write your pallas kernel here")


# speedup vs baseline: 7.9447x; 7.9447x over previous
import functools

import jax
import jax.numpy as jnp
from jax import lax
from jax.experimental import pallas as pl
from jax.experimental.pallas import tpu as pltpu
from jax.experimental.pallas import tpu_sc as plsc

_N_HEADS = 8
_MAX_DIST = 512

_NC = 2
_NS = 16
_NW = _NC * _NS
_LANES = 16

_CHUNK = 2048


def _body(dist_hbm, table_hbm, out_hbm, table_v, idx_v, out_v):
    wid = lax.axis_index("s") * _NC + lax.axis_index("c")
    n_total = dist_hbm.shape[0]
    per_w = n_total // _NW
    n_chunks = per_w // _CHUNK
    base = wid * per_w

    pltpu.sync_copy(table_hbm, table_v)

    iota = lax.iota(jnp.int32, _LANES)
    col = lax.bitwise_and(iota, _N_HEADS - 1)
    half = lax.shift_right_logical(iota, 3)

    def chunk_step(g, carry):
        off = base + g * _CHUNK
        pltpu.sync_copy(dist_hbm.at[pl.ds(off, _CHUNK)], idx_v)

        def group_step(i, carry2):
            ivec = idx_v[pl.ds(i * _LANES, _LANES)]
            rows = jnp.minimum(jnp.maximum(ivec, -1), _MAX_DIST) + 1
            obase = i * (_LANES * _N_HEADS)
            for v in range(_N_HEADS):
                sel = half + (2 * v)
                rep = lax.gather(
                    rows,
                    sel[:, None],
                    lax.GatherDimensionNumbers(
                        offset_dims=(),
                        collapsed_slice_dims=(0,),
                        start_index_map=(0,),
                    ),
                    (1,),
                    mode=lax.GatherScatterMode.PROMISE_IN_BOUNDS,
                )
                addr = lax.shift_left(rep, 3) + col
                vals = plsc.load_gather(table_v, [addr])
                out_v[pl.ds(obase + v * _LANES, _LANES)] = vals
            return carry2

        lax.fori_loop(0, _CHUNK // _LANES, group_step, 0, unroll=2)
        pltpu.sync_copy(out_v, out_hbm.at[pl.ds(off * _N_HEADS, _CHUNK * _N_HEADS)])
        return carry

    lax.fori_loop(0, n_chunks, chunk_step, 0)


def kernel(dist, table):
    n = dist.size
    dist_flat = dist.reshape(n)

    mesh = plsc.VectorSubcoreMesh(core_axis_name="c", subcore_axis_name="s")
    run = pl.kernel(
        _body,
        out_type=jax.ShapeDtypeStruct((n * _N_HEADS,), jnp.float32),
        mesh=mesh,
        compiler_params=pltpu.CompilerParams(needs_layout_passes=False),
        scratch_types=[
            pltpu.VMEM(((_MAX_DIST + 2) * _N_HEADS,), jnp.float32),
            pltpu.VMEM((_CHUNK,), jnp.int32),
            pltpu.VMEM((_CHUNK * _N_HEADS,), jnp.float32),
        ],
    )
    out_flat = run(dist_flat, table.reshape(-1))
    return out_flat.reshape(*dist.shape, _N_HEADS)


# speedup vs baseline: 27.4141x; 3.4506x over previous
import jax
import jax.numpy as jnp
from jax import lax
from jax.experimental import pallas as pl
from jax.experimental.pallas import tpu as pltpu
from jax.experimental.pallas import tpu_sc as plsc

_N_HEADS = 8
_MAX_DIST = 512

_NC = 2
_NS = 16
_NW = _NC * _NS
_LANES = 16

_SLAB = 4096
_OUT_SLAB = _SLAB * _N_HEADS


def _body(dist_hbm, table_hbm, out_hbm, table_v, idx_v, out_v):
    wid = lax.axis_index("s") * _NC + lax.axis_index("c")
    n_slabs = dist_hbm.shape[0] // _SLAB
    slabs_per_w = n_slabs // _NW
    s0 = wid * slabs_per_w

    pltpu.sync_copy(table_hbm, table_v)

    def slab_step(s, carry):
        in_off = (s0 + s) * _SLAB
        pltpu.sync_copy(dist_hbm.at[pl.ds(in_off, _SLAB)], idx_v)

        def group_step(g, carry2):
            jt = lax.shift_right_logical(g, 6)
            il = lax.bitwise_and(lax.shift_right_logical(g, 3), 7)
            k16 = lax.shift_left(lax.bitwise_and(g, 7), 4)
            in_base = lax.shift_left(jt * 8 + il, 7) + k16
            out_base = lax.shift_left(il * 4 + jt, 10) + k16
            ivec = idx_v[pl.ds(in_base, _LANES)]
            rows8 = lax.shift_left(
                jnp.minimum(jnp.maximum(ivec, -1), _MAX_DIST) + 1, 3
            )
            for h in range(_N_HEADS):
                vals = plsc.load_gather(table_v, [rows8 + h])
                out_v[pl.ds(out_base + h * 128, _LANES)] = vals
            return carry2

        lax.fori_loop(0, _SLAB // _LANES, group_step, 0, unroll=2)
        pltpu.sync_copy(out_v, out_hbm.at[pl.ds(in_off * _N_HEADS, _OUT_SLAB)])
        return carry

    lax.fori_loop(0, slabs_per_w, slab_step, 0)


def kernel(dist, table):
    b, n_i, n_j = dist.shape
    n = dist.size
    dist_flat = (
        dist.reshape(b, n_i // 8, 8, n_j // 128, 128)
        .transpose(0, 1, 3, 2, 4)
        .reshape(n)
    )

    mesh = plsc.VectorSubcoreMesh(core_axis_name="c", subcore_axis_name="s")
    run = pl.kernel(
        _body,
        out_type=jax.ShapeDtypeStruct((n * _N_HEADS,), jnp.float32),
        mesh=mesh,
        compiler_params=pltpu.CompilerParams(needs_layout_passes=False),
        scratch_types=[
            pltpu.VMEM(((_MAX_DIST + 2) * _N_HEADS,), jnp.float32),
            pltpu.VMEM((_SLAB,), jnp.int32),
            pltpu.VMEM((_OUT_SLAB,), jnp.float32),
        ],
    )
    out_flat = run(dist_flat, table.reshape(-1))
    out = (
        out_flat.reshape(b, n_i, n_j // 128, _N_HEADS, 128)
        .transpose(0, 1, 2, 4, 3)
        .reshape(b, n_i, n_j, _N_HEADS)
    )
    return out


# speedup vs baseline: 45.7793x; 1.6699x over previous
import jax
import jax.numpy as jnp
from jax import lax
from jax.experimental import pallas as pl
from jax.experimental.pallas import tpu as pltpu
from jax.experimental.pallas import tpu_sc as plsc

_N_HEADS = 8
_MAX_DIST = 512

_NC = 2
_NS = 16
_NW = _NC * _NS
_LANES = 16

_SLAB = 4096
_HALF_OUT = _SLAB * _N_HEADS // 2
_TROWS = _MAX_DIST + 2
_TWORDS = _TROWS * _N_HEADS
_TSTRIDE = _TWORDS + 1


def _body(dist_hbm, table_hbm, out_hbm, table16, idx0, idx1, out0, out1,
          isem0, isem1, osem0, osem1):
    wid = lax.axis_index("s") * _NC + lax.axis_index("c")
    n_slabs = dist_hbm.shape[0] // _SLAB
    per_w = n_slabs // _NW
    s0 = wid * per_w

    iota = lax.iota(jnp.int32, _LANES)
    loff = iota * _TSTRIDE

    pltpu.sync_copy(table_hbm, out0.at[pl.ds(0, _TWORDS)])

    def fill_step(w, carry):
        vals = out0[pl.ds(w * _LANES, _LANES)]
        base = iota + w * _LANES
        for c in range(_LANES):
            plsc.store_scatter(table16, [base + c * _TSTRIDE], vals)
        return carry

    lax.fori_loop(0, _TWORDS // _LANES, fill_step, 0)

    idx_bufs = (idx0, idx1)
    isems = (isem0, isem1)
    out_bufs = (out0, out1)
    osems = (osem0, osem1)

    def idx_copy(s, p):
        return pltpu.make_async_copy(
            dist_hbm.at[pl.ds((s0 + s) * _SLAB, _SLAB)], idx_bufs[p], isems[p]
        )

    def out_copy(s, q, r):
        off = (s0 + s) * _SLAB * _N_HEADS + q * _HALF_OUT
        return pltpu.make_async_copy(
            out_bufs[r], out_hbm.at[pl.ds(off, _HALF_OUT)], osems[r]
        )

    def out_drain(r):
        pltpu.make_async_copy(
            out_bufs[r], out_hbm.at[pl.ds(0, _HALF_OUT)], osems[r]
        ).wait()

    def compute_half(idx_v, out_v, q):
        def group_step(g, carry):
            jt = lax.shift_right_logical(g, 5)
            il_rel = lax.bitwise_and(lax.shift_right_logical(g, 3), 3)
            k16 = lax.shift_left(lax.bitwise_and(g, 7), 4)
            in_base = lax.shift_left(jt * 8 + (il_rel + 4 * q), 7) + k16
            out_base = lax.shift_left(il_rel * 4 + jt, 10) + k16
            ivec = idx_v[pl.ds(in_base, _LANES)]
            addr = loff + lax.shift_left(
                jnp.minimum(jnp.maximum(ivec, -1), _MAX_DIST) + 1, 3
            )
            for h in range(_N_HEADS):
                vals = plsc.load_gather(table16, [addr + h])
                out_v[pl.ds(out_base + h * 128, _LANES)] = vals
            return carry

        lax.fori_loop(0, _SLAB // _LANES // 2, group_step, 0, unroll=4)

    idx_copy(0, 0).start()

    @pl.loop(0, per_w, step=2)
    def slab_pair(s2):
        pltpu.make_async_copy(
            dist_hbm.at[pl.ds(0, _SLAB)], idx_bufs[0], isems[0]
        ).wait()
        idx_copy(s2 + 1, 1).start()
        for q in range(2):
            @pl.when(s2 > 0)
            def _():
                out_drain(q)
            compute_half(idx_bufs[0], out_bufs[q], q)
            out_copy(s2, q, q).start()
        pltpu.make_async_copy(
            dist_hbm.at[pl.ds(0, _SLAB)], idx_bufs[1], isems[1]
        ).wait()

        @pl.when(s2 + 2 < per_w)
        def _():
            idx_copy(s2 + 2, 0).start()

        for q in range(2):
            out_drain(q)
            compute_half(idx_bufs[1], out_bufs[q], q)
            out_copy(s2 + 1, q, q).start()

    out_drain(0)
    out_drain(1)


def kernel(dist, table):
    b, n_i, n_j = dist.shape
    n = dist.size
    dist_flat = (
        dist.reshape(b, n_i // 8, 8, n_j // 128, 128)
        .transpose(0, 1, 3, 2, 4)
        .reshape(n)
    )

    mesh = plsc.VectorSubcoreMesh(core_axis_name="c", subcore_axis_name="s")
    run = pl.kernel(
        _body,
        out_type=jax.ShapeDtypeStruct((n * _N_HEADS,), jnp.float32),
        mesh=mesh,
        compiler_params=pltpu.CompilerParams(needs_layout_passes=False),
        scratch_types=[
            pltpu.VMEM((_TSTRIDE * _LANES,), jnp.float32),
            pltpu.VMEM((_SLAB,), jnp.int32),
            pltpu.VMEM((_SLAB,), jnp.int32),
            pltpu.VMEM((_HALF_OUT,), jnp.float32),
            pltpu.VMEM((_HALF_OUT,), jnp.float32),
            pltpu.SemaphoreType.DMA,
            pltpu.SemaphoreType.DMA,
            pltpu.SemaphoreType.DMA,
            pltpu.SemaphoreType.DMA,
        ],
    )
    out_flat = run(dist_flat, table.reshape(-1))
    out = (
        out_flat.reshape(b, n_i, n_j // 128, _N_HEADS, 128)
        .transpose(0, 1, 2, 4, 3)
        .reshape(b, n_i, n_j, _N_HEADS)
    )
    return out


# speedup vs baseline: 261.3841x; 5.7097x over previous
import jax
import jax.numpy as jnp
from jax import lax
from jax.experimental import pallas as pl
from jax.experimental.pallas import tpu as pltpu
from jax.experimental.pallas import tpu_sc as plsc

_N_HEADS = 8
_MAX_DIST = 512

_NC = 2
_NS = 16
_NW = _NC * _NS
_LANES = 16

_SLAB = 4096
_HALF_OUT = _SLAB * _N_HEADS // 2
_TROWS = _MAX_DIST + 2
_TWORDS = _TROWS * _N_HEADS
_TSTRIDE = _TWORDS + 1


def _body(dist_hbm, table_hbm, out_hbm, table16, idx0, idx1, out0, out1,
          isem0, isem1, osem0, osem1):
    wid = lax.axis_index("s") * _NC + lax.axis_index("c")
    n_slabs = dist_hbm.shape[0] // _SLAB
    per_w = n_slabs // _NW
    s0 = wid * per_w

    iota = lax.iota(jnp.int32, _LANES)
    loff = iota * _TSTRIDE

    pltpu.sync_copy(table_hbm, out0.at[pl.ds(0, _TWORDS)])

    def fill_step(w, carry):
        vals = out0[pl.ds(w * _LANES, _LANES)]
        base = iota + w * _LANES
        for c in range(_LANES):
            plsc.store_scatter(table16, [base + c * _TSTRIDE], vals)
        return carry

    lax.fori_loop(0, _TWORDS // _LANES, fill_step, 0)

    idx_bufs = (idx0, idx1)
    isems = (isem0, isem1)
    out_bufs = (out0, out1)
    osems = (osem0, osem1)

    def idx_copy(s, p):
        return pltpu.make_async_copy(
            dist_hbm.at[pl.ds((s0 + s) * _SLAB, _SLAB)], idx_bufs[p], isems[p]
        )

    def out_copy(s, q, r):
        off = (s0 + s) * _SLAB * _N_HEADS + q * _HALF_OUT
        return pltpu.make_async_copy(
            out_bufs[r], out_hbm.at[pl.ds(off, _HALF_OUT)], osems[r]
        )

    def out_drain(r):
        pltpu.make_async_copy(
            out_bufs[r], out_hbm.at[pl.ds(0, _HALF_OUT)], osems[r]
        ).wait()

    def compute_half(idx_v, out_v, q):
        @plsc.parallel_loop(0, _SLAB // _LANES // 2, unroll=4)
        def group_step(g):
            jt = lax.shift_right_logical(g, 5)
            il_rel = lax.bitwise_and(lax.shift_right_logical(g, 3), 3)
            k16 = lax.shift_left(lax.bitwise_and(g, 7), 4)
            in_base = lax.shift_left(jt * 8 + (il_rel + 4 * q), 7) + k16
            out_base = lax.shift_left(il_rel * 4 + jt, 10) + k16
            ivec = idx_v[pl.ds(in_base, _LANES)]
            addr = loff + lax.shift_left(
                jnp.minimum(jnp.maximum(ivec, -1), _MAX_DIST) + 1, 3
            )
            for h in range(_N_HEADS):
                vals = plsc.load_gather(table16, [addr + h])
                out_v[pl.ds(out_base + h * 128, _LANES)] = vals

    idx_copy(0, 0).start()

    @pl.loop(0, per_w, step=2)
    def slab_pair(s2):
        pltpu.make_async_copy(
            dist_hbm.at[pl.ds(0, _SLAB)], idx_bufs[0], isems[0]
        ).wait()
        idx_copy(s2 + 1, 1).start()
        for q in range(2):
            @pl.when(s2 > 0)
            def _():
                out_drain(q)
            compute_half(idx_bufs[0], out_bufs[q], q)
            out_copy(s2, q, q).start()
        pltpu.make_async_copy(
            dist_hbm.at[pl.ds(0, _SLAB)], idx_bufs[1], isems[1]
        ).wait()

        @pl.when(s2 + 2 < per_w)
        def _():
            idx_copy(s2 + 2, 0).start()

        for q in range(2):
            out_drain(q)
            compute_half(idx_bufs[1], out_bufs[q], q)
            out_copy(s2 + 1, q, q).start()

    out_drain(0)
    out_drain(1)


def kernel(dist, table):
    b, n_i, n_j = dist.shape
    n = dist.size
    dist_flat = (
        dist.reshape(b, n_i // 8, 8, n_j // 128, 128)
        .transpose(0, 1, 3, 2, 4)
        .reshape(n)
    )

    mesh = plsc.VectorSubcoreMesh(core_axis_name="c", subcore_axis_name="s")
    run = pl.kernel(
        _body,
        out_type=jax.ShapeDtypeStruct((n * _N_HEADS,), jnp.float32),
        mesh=mesh,
        compiler_params=pltpu.CompilerParams(needs_layout_passes=False),
        scratch_types=[
            pltpu.VMEM((_TSTRIDE * _LANES,), jnp.float32),
            pltpu.VMEM((_SLAB,), jnp.int32),
            pltpu.VMEM((_SLAB,), jnp.int32),
            pltpu.VMEM((_HALF_OUT,), jnp.float32),
            pltpu.VMEM((_HALF_OUT,), jnp.float32),
            pltpu.SemaphoreType.DMA,
            pltpu.SemaphoreType.DMA,
            pltpu.SemaphoreType.DMA,
            pltpu.SemaphoreType.DMA,
        ],
    )
    out_flat = run(dist_flat, table.reshape(-1))
    out = (
        out_flat.reshape(b, n_i, n_j // 128, _N_HEADS, 128)
        .transpose(0, 1, 2, 4, 3)
        .reshape(b, n_i, n_j, _N_HEADS)
    )
    return out
